# Initial kernel scaffold; baseline (speedup 1.0000x reference)
import jax
import jax.numpy as jnp
from jax.experimental import pallas as pl


def kernel(x, edge_index, W1l, W1r, b1, W2l, W2r, b2):
    raise NotImplementedError("write your pallas kernel here")


# speedup vs baseline: 8.9608x; 8.9608x over previous
import functools

import jax
import jax.numpy as jnp
from jax import lax
from jax.experimental import pallas as pl
from jax.experimental.pallas import tpu as pltpu
from jax.experimental.pallas import tpu_sc as plsc

_NC = 2
_NS = 16
_NW = _NC * _NS

_C = 125
_G = 10


def _seg_sum_sc(d_model, n_nodes, n_chunks, with_counts):
  rpt = n_nodes // _NS
  n_groups = n_chunks // _G
  mesh = plsc.VectorSubcoreMesh(core_axis_name="c", subcore_axis_name="s")

  out_type = [jax.ShapeDtypeStruct((_NC, n_nodes, d_model), jnp.float32)]
  scratch = [
      pltpu.VMEM_SHARED((n_nodes, d_model), jnp.float32),
      pltpu.VMEM((_G, _C), jnp.int32),
      pltpu.VMEM((_G, _C), jnp.int32),
      pltpu.VMEM((_C, d_model), jnp.float32),
      pltpu.SemaphoreType.DMA,
  ]
  if with_counts:
    out_type.append(jax.ShapeDtypeStruct((_NC, n_nodes, 16), jnp.float32))
    scratch += [
        pltpu.VMEM_SHARED((n_nodes, 16), jnp.float32),
        pltpu.VMEM((_C, 16), jnp.float32),
    ]

  def body(*refs):
    if with_counts:
      (y, src, dst, ones, zrow, zcnt, seg_out, cnt_out,
       acc, src_v, dst_v, rows_v, sem, cnt_sh, ones_v) = refs
    else:
      (y, src, dst, zrow, seg_out,
       acc, src_v, dst_v, rows_v, sem) = refs
    cid = lax.axis_index("c")
    sid = lax.axis_index("s")
    wid = sid * _NC + cid

    pltpu.sync_copy(zrow, acc.at[pl.ds(sid * rpt, rpt)])
    if with_counts:
      pltpu.sync_copy(ones, ones_v)
      pltpu.sync_copy(zcnt, cnt_sh.at[pl.ds(sid * rpt, rpt)])
    plsc.subcore_barrier()

    def group(g, carry):
      pltpu.sync_copy(src.at[wid, pl.ds(g * _G, _G)], src_v)
      pltpu.sync_copy(dst.at[wid, pl.ds(g * _G, _G)], dst_v)

      def chunk(j, carry2):
        pltpu.async_copy(y.at[src_v.at[j]], rows_v, sem).wait()
        pltpu.sync_copy(rows_v, acc.at[dst_v.at[j]], add=True)
        if with_counts:
          pltpu.sync_copy(ones_v, cnt_sh.at[dst_v.at[j]], add=True)
        return carry2

      return lax.fori_loop(0, _G, chunk, carry)

    lax.fori_loop(0, n_groups, group, 0)
    plsc.subcore_barrier()

    wb = (n_nodes // _NS) // 8 * 8
    tail = n_nodes - _NS * wb
    pltpu.sync_copy(acc.at[pl.ds(sid * wb, wb)],
                    seg_out.at[cid, pl.ds(sid * wb, wb)])
    if with_counts:
      pltpu.sync_copy(cnt_sh.at[pl.ds(sid * wb, wb)],
                      cnt_out.at[cid, pl.ds(sid * wb, wb)])
    if tail:
      @pl.when(sid == 0)
      def _():
        pltpu.sync_copy(acc.at[pl.ds(_NS * wb, tail)],
                        seg_out.at[cid, pl.ds(_NS * wb, tail)])
        if with_counts:
          pltpu.sync_copy(cnt_sh.at[pl.ds(_NS * wb, tail)],
                          cnt_out.at[cid, pl.ds(_NS * wb, tail)])

  return pl.kernel(
      body, out_type=out_type, mesh=mesh, scratch_types=scratch,
      compiler_params=pltpu.CompilerParams(use_tc_tiling_on_sc=False))


def _proj2(x, Wa, Wb, b, block_n):
  n, d_in = x.shape
  d_out = Wa.shape[1]

  def body(x_ref, wa_ref, wb_ref, b_ref, ya_ref, yb_ref):
    xb = x_ref[...]
    ya_ref[...] = jnp.dot(xb, wa_ref[...], preferred_element_type=jnp.float32)
    yb_ref[...] = (jnp.dot(xb, wb_ref[...], preferred_element_type=jnp.float32)
                   + b_ref[...])

  return pl.pallas_call(
      body,
      grid=(n // block_n,),
      in_specs=[
          pl.BlockSpec((block_n, d_in), lambda i: (i, 0)),
          pl.BlockSpec((d_in, d_out), lambda i: (0, 0)),
          pl.BlockSpec((d_in, d_out), lambda i: (0, 0)),
          pl.BlockSpec((1, d_out), lambda i: (0, 0)),
      ],
      out_specs=[
          pl.BlockSpec((block_n, d_out), lambda i: (i, 0)),
          pl.BlockSpec((block_n, d_out), lambda i: (i, 0)),
      ],
      out_shape=[
          jax.ShapeDtypeStruct((n, d_out), jnp.float32),
          jax.ShapeDtypeStruct((n, d_out), jnp.float32),
      ],
  )(x, Wa, Wb, b.reshape(1, -1))


def _mid_layer(seg, cnt, z1, W2l, W2r, b2, block_n):
  _, n, d_h = seg.shape
  d_out = W2l.shape[1]

  def body(s_ref, c_ref, z1_ref, wl_ref, wr_ref, b_ref, y2_ref, z2_ref):
    c = jnp.maximum(c_ref[0, :, :1] + c_ref[1, :, :1], 1.0)
    mean = (s_ref[0] + s_ref[1]) / c
    h = jnp.maximum(mean + z1_ref[...], 0.0)
    y2_ref[...] = jnp.dot(h, wl_ref[...], preferred_element_type=jnp.float32)
    z2_ref[...] = (jnp.dot(h, wr_ref[...], preferred_element_type=jnp.float32)
                   + b_ref[...])

  return pl.pallas_call(
      body,
      grid=(n // block_n,),
      in_specs=[
          pl.BlockSpec((2, block_n, d_h), lambda i: (0, i, 0)),
          pl.BlockSpec((2, block_n, 16), lambda i: (0, i, 0)),
          pl.BlockSpec((block_n, d_h), lambda i: (i, 0)),
          pl.BlockSpec((d_h, d_out), lambda i: (0, 0)),
          pl.BlockSpec((d_h, d_out), lambda i: (0, 0)),
          pl.BlockSpec((1, d_out), lambda i: (0, 0)),
      ],
      out_specs=[
          pl.BlockSpec((block_n, d_out), lambda i: (i, 0)),
          pl.BlockSpec((block_n, d_out), lambda i: (i, 0)),
      ],
      out_shape=[
          jax.ShapeDtypeStruct((n, d_out), jnp.float32),
          jax.ShapeDtypeStruct((n, d_out), jnp.float32),
      ],
  )(seg, cnt, z1, W2l, W2r, b2.reshape(1, -1))


def _final_layer(seg, cnt, z2, block_n):
  _, n, d_out = seg.shape

  def body(s_ref, c_ref, z2_ref, out_ref):
    c = jnp.maximum(c_ref[0, :, :1] + c_ref[1, :, :1], 1.0)
    v = (s_ref[0] + s_ref[1]) / c + z2_ref[...]
    m = jnp.max(v, axis=1, keepdims=True)
    e = jnp.exp(v - m)
    s = jnp.sum(e, axis=1, keepdims=True)
    out_ref[...] = v - m - jnp.log(s)

  return pl.pallas_call(
      body,
      grid=(n // block_n,),
      in_specs=[
          pl.BlockSpec((2, block_n, d_out), lambda i: (0, i, 0)),
          pl.BlockSpec((2, block_n, 16), lambda i: (0, i, 0)),
          pl.BlockSpec((block_n, d_out), lambda i: (i, 0)),
      ],
      out_specs=pl.BlockSpec((block_n, d_out), lambda i: (i, 0)),
      out_shape=jax.ShapeDtypeStruct((n, d_out), jnp.float32),
  )(seg, cnt, z2)


def kernel(x, edge_index, W1l, W1r, b1, W2l, W2r, b2):
  n, d_in = x.shape
  e = edge_index.shape[1]
  d_h = W1l.shape[1]
  d_out = W2l.shape[1]

  epw = e // _NW
  n_chunks = epw // _C
  block_n = 1000

  src = edge_index[0].reshape(_NW, n_chunks, _C)
  dst = edge_index[1].reshape(_NW, n_chunks, _C)
  ones = jnp.ones((_C, 16), jnp.float32)
  zrow_h = jnp.zeros((n // _NS, d_h), jnp.float32)
  zrow_o = jnp.zeros((n // _NS, d_out), jnp.float32)
  zcnt = jnp.zeros((n // _NS, 16), jnp.float32)

  y1, z1 = _proj2(x, W1l, W1r, b1, block_n)
  seg1, cnt = _seg_sum_sc(d_h, n, n_chunks, True)(
      y1, src, dst, ones, zrow_h, zcnt)
  y2, z2 = _mid_layer(seg1, cnt, z1, W2l, W2r, b2, block_n)

  (seg2,) = _seg_sum_sc(d_out, n, n_chunks, False)(y2, src, dst, zrow_o)
  return _final_layer(seg2, cnt, z2, block_n)


# speedup vs baseline: 11.9615x; 1.3349x over previous
import functools

import jax
import jax.numpy as jnp
from jax import lax
from jax.experimental import pallas as pl
from jax.experimental.pallas import tpu as pltpu
from jax.experimental.pallas import tpu_sc as plsc

_NC = 2
_NS = 16
_NW = _NC * _NS

_C = 125
_G = 10


def _seg_sum_sc(d_model, n_nodes, n_chunks, with_counts):
  rpt = n_nodes // _NS
  n_groups = n_chunks // _G
  mesh = plsc.VectorSubcoreMesh(core_axis_name="c", subcore_axis_name="s")

  out_type = [jax.ShapeDtypeStruct((_NC, n_nodes, d_model), jnp.float32)]
  scratch = [
      pltpu.VMEM_SHARED((n_nodes, d_model), jnp.float32),
      pltpu.VMEM((_G, _C), jnp.int32),
      pltpu.VMEM((_G, _C), jnp.int32),
      pltpu.VMEM((_C, d_model), jnp.float32),
      pltpu.VMEM((_C, d_model), jnp.float32),
      pltpu.SemaphoreType.DMA,
      pltpu.SemaphoreType.DMA,
  ]
  if with_counts:
    out_type.append(jax.ShapeDtypeStruct((_NC, n_nodes, 16), jnp.float32))
    scratch += [
        pltpu.VMEM_SHARED((n_nodes, 16), jnp.float32),
        pltpu.VMEM((_C, 16), jnp.float32),
    ]

  def body(*refs):
    if with_counts:
      (y, src, dst, ones, zrow, zcnt, seg_out, cnt_out,
       acc, src_v, dst_v, rows0, rows1, sem0, sem1, cnt_sh, ones_v) = refs
    else:
      (y, src, dst, zrow, seg_out,
       acc, src_v, dst_v, rows0, rows1, sem0, sem1) = refs
    rows = (rows0, rows1)
    sems = (sem0, sem1)
    cid = lax.axis_index("c")
    sid = lax.axis_index("s")
    wid = sid * _NC + cid

    pltpu.sync_copy(zrow, acc.at[pl.ds(sid * rpt, rpt)])
    if with_counts:
      pltpu.sync_copy(ones, ones_v)
      pltpu.sync_copy(zcnt, cnt_sh.at[pl.ds(sid * rpt, rpt)])
    plsc.subcore_barrier()

    def group(g, carry):
      pltpu.sync_copy(src.at[wid, pl.ds(g * _G, _G)], src_v)
      pltpu.sync_copy(dst.at[wid, pl.ds(g * _G, _G)], dst_v)

      pending = pltpu.async_copy(y.at[src_v.at[0]], rows[0], sems[0])
      for j in range(_G):
        p = j % 2
        if j + 1 < _G:
          nxt = pltpu.async_copy(y.at[src_v.at[j + 1]], rows[1 - p],
                                 sems[1 - p])
        pending.wait()
        pltpu.sync_copy(rows[p], acc.at[dst_v.at[j]], add=True)
        if with_counts:
          pltpu.sync_copy(ones_v, cnt_sh.at[dst_v.at[j]], add=True)
        if j + 1 < _G:
          pending = nxt
      return carry

    lax.fori_loop(0, n_groups, group, 0)
    plsc.subcore_barrier()

    wb = (n_nodes // _NS) // 8 * 8
    tail = n_nodes - _NS * wb
    pltpu.sync_copy(acc.at[pl.ds(sid * wb, wb)],
                    seg_out.at[cid, pl.ds(sid * wb, wb)])
    if with_counts:
      pltpu.sync_copy(cnt_sh.at[pl.ds(sid * wb, wb)],
                      cnt_out.at[cid, pl.ds(sid * wb, wb)])
    if tail:
      @pl.when(sid == 0)
      def _():
        pltpu.sync_copy(acc.at[pl.ds(_NS * wb, tail)],
                        seg_out.at[cid, pl.ds(_NS * wb, tail)])
        if with_counts:
          pltpu.sync_copy(cnt_sh.at[pl.ds(_NS * wb, tail)],
                          cnt_out.at[cid, pl.ds(_NS * wb, tail)])

  return pl.kernel(
      body, out_type=out_type, mesh=mesh, scratch_types=scratch,
      compiler_params=pltpu.CompilerParams(use_tc_tiling_on_sc=False))


def _proj2(x, Wa, Wb, b, block_n):
  n, d_in = x.shape
  d_out = Wa.shape[1]

  def body(x_ref, wa_ref, wb_ref, b_ref, ya_ref, yb_ref):
    xb = x_ref[...]
    ya_ref[...] = jnp.dot(xb, wa_ref[...], preferred_element_type=jnp.float32)
    yb_ref[...] = (jnp.dot(xb, wb_ref[...], preferred_element_type=jnp.float32)
                   + b_ref[...])

  return pl.pallas_call(
      body,
      grid=(n // block_n,),
      in_specs=[
          pl.BlockSpec((block_n, d_in), lambda i: (i, 0)),
          pl.BlockSpec((d_in, d_out), lambda i: (0, 0)),
          pl.BlockSpec((d_in, d_out), lambda i: (0, 0)),
          pl.BlockSpec((1, d_out), lambda i: (0, 0)),
      ],
      out_specs=[
          pl.BlockSpec((block_n, d_out), lambda i: (i, 0)),
          pl.BlockSpec((block_n, d_out), lambda i: (i, 0)),
      ],
      out_shape=[
          jax.ShapeDtypeStruct((n, d_out), jnp.float32),
          jax.ShapeDtypeStruct((n, d_out), jnp.float32),
      ],
  )(x, Wa, Wb, b.reshape(1, -1))


def _mid_layer(seg, cnt, z1, W2l, W2r, b2, block_n):
  _, n, d_h = seg.shape
  d_out = W2l.shape[1]

  def body(s_ref, c_ref, z1_ref, wl_ref, wr_ref, b_ref, y2_ref, z2_ref):
    c = jnp.maximum(c_ref[0, :, :1] + c_ref[1, :, :1], 1.0)
    mean = (s_ref[0] + s_ref[1]) / c
    h = jnp.maximum(mean + z1_ref[...], 0.0)
    y2_ref[...] = jnp.dot(h, wl_ref[...], preferred_element_type=jnp.float32)
    z2_ref[...] = (jnp.dot(h, wr_ref[...], preferred_element_type=jnp.float32)
                   + b_ref[...])

  return pl.pallas_call(
      body,
      grid=(n // block_n,),
      in_specs=[
          pl.BlockSpec((2, block_n, d_h), lambda i: (0, i, 0)),
          pl.BlockSpec((2, block_n, 16), lambda i: (0, i, 0)),
          pl.BlockSpec((block_n, d_h), lambda i: (i, 0)),
          pl.BlockSpec((d_h, d_out), lambda i: (0, 0)),
          pl.BlockSpec((d_h, d_out), lambda i: (0, 0)),
          pl.BlockSpec((1, d_out), lambda i: (0, 0)),
      ],
      out_specs=[
          pl.BlockSpec((block_n, d_out), lambda i: (i, 0)),
          pl.BlockSpec((block_n, d_out), lambda i: (i, 0)),
      ],
      out_shape=[
          jax.ShapeDtypeStruct((n, d_out), jnp.float32),
          jax.ShapeDtypeStruct((n, d_out), jnp.float32),
      ],
  )(seg, cnt, z1, W2l, W2r, b2.reshape(1, -1))


def _final_layer(seg, cnt, z2, block_n):
  _, n, d_out = seg.shape

  def body(s_ref, c_ref, z2_ref, out_ref):
    c = jnp.maximum(c_ref[0, :, :1] + c_ref[1, :, :1], 1.0)
    v = (s_ref[0] + s_ref[1]) / c + z2_ref[...]
    m = jnp.max(v, axis=1, keepdims=True)
    e = jnp.exp(v - m)
    s = jnp.sum(e, axis=1, keepdims=True)
    out_ref[...] = v - m - jnp.log(s)

  return pl.pallas_call(
      body,
      grid=(n // block_n,),
      in_specs=[
          pl.BlockSpec((2, block_n, d_out), lambda i: (0, i, 0)),
          pl.BlockSpec((2, block_n, 16), lambda i: (0, i, 0)),
          pl.BlockSpec((block_n, d_out), lambda i: (i, 0)),
      ],
      out_specs=pl.BlockSpec((block_n, d_out), lambda i: (i, 0)),
      out_shape=jax.ShapeDtypeStruct((n, d_out), jnp.float32),
  )(seg, cnt, z2)


def kernel(x, edge_index, W1l, W1r, b1, W2l, W2r, b2):
  n, d_in = x.shape
  e = edge_index.shape[1]
  d_h = W1l.shape[1]
  d_out = W2l.shape[1]

  epw = e // _NW
  n_chunks = epw // _C
  block_n = 1000

  src = edge_index[0].reshape(_NW, n_chunks, _C)
  dst = edge_index[1].reshape(_NW, n_chunks, _C)
  ones = jnp.ones((_C, 16), jnp.float32)
  zrow_h = jnp.zeros((n // _NS, d_h), jnp.float32)
  zrow_o = jnp.zeros((n // _NS, d_out), jnp.float32)
  zcnt = jnp.zeros((n // _NS, 16), jnp.float32)

  y1, z1 = _proj2(x, W1l, W1r, b1, block_n)
  seg1, cnt = _seg_sum_sc(d_h, n, n_chunks, True)(
      y1, src, dst, ones, zrow_h, zcnt)
  y2, z2 = _mid_layer(seg1, cnt, z1, W2l, W2r, b2, block_n)

  (seg2,) = _seg_sum_sc(d_out, n, n_chunks, False)(y2, src, dst, zrow_o)
  return _final_layer(seg2, cnt, z2, block_n)


# speedup vs baseline: 12.4538x; 1.0412x over previous
import functools

import jax
import jax.numpy as jnp
from jax import lax
from jax.experimental import pallas as pl
from jax.experimental.pallas import tpu as pltpu
from jax.experimental.pallas import tpu_sc as plsc

_NC = 2
_NS = 16
_NW = _NC * _NS

_C = 125
_G = 16


def _seg_sum_sc(d_model, n_nodes, n_chunks, with_counts):
  rpt = n_nodes // _NS
  n_groups = n_chunks // _G
  mesh = plsc.VectorSubcoreMesh(core_axis_name="c", subcore_axis_name="s")

  out_type = [jax.ShapeDtypeStruct((_NC, n_nodes, d_model), jnp.float32)]
  scratch = [
      pltpu.VMEM_SHARED((n_nodes, d_model), jnp.float32),
      pltpu.VMEM((_G, _C), jnp.int32),
      pltpu.VMEM((_G, _C), jnp.int32),
      pltpu.VMEM((_C, d_model), jnp.float32),
      pltpu.VMEM((_C, d_model), jnp.float32),
      pltpu.SemaphoreType.DMA,
      pltpu.SemaphoreType.DMA,
      pltpu.SemaphoreType.DMA,
      pltpu.SemaphoreType.DMA,
  ]
  if with_counts:
    out_type.append(jax.ShapeDtypeStruct((_NC, n_nodes, 16), jnp.float32))
    scratch += [
        pltpu.VMEM_SHARED((n_nodes, 16), jnp.float32),
        pltpu.VMEM((_C, 16), jnp.float32),
        pltpu.SemaphoreType.DMA,
    ]

  def body(*refs):
    if with_counts:
      (y, src, dst, ones, zrow, zcnt, seg_out, cnt_out,
       acc, src_v, dst_v, rows0, rows1, gs0, gs1, ss0, ss1,
       cnt_sh, ones_v, cs) = refs
    else:
      (y, src, dst, zrow, seg_out,
       acc, src_v, dst_v, rows0, rows1, gs0, gs1, ss0, ss1) = refs
    rows = (rows0, rows1)
    gsem = (gs0, gs1)
    ssem = (ss0, ss1)
    cid = lax.axis_index("c")
    sid = lax.axis_index("s")
    wid = sid * _NC + cid

    pltpu.sync_copy(zrow, acc.at[pl.ds(sid * rpt, rpt)])
    if with_counts:
      pltpu.sync_copy(ones, ones_v)
      pltpu.sync_copy(zcnt, cnt_sh.at[pl.ds(sid * rpt, rpt)])
    plsc.subcore_barrier()

    def group(g, carry):
      pltpu.sync_copy(src.at[wid, pl.ds(g * _G, _G)], src_v)
      pltpu.sync_copy(dst.at[wid, pl.ds(g * _G, _G)], dst_v)

      gath = [None, None]
      scat = [None, None]
      cnt_cp = None
      gath[0] = pltpu.async_copy(y.at[src_v.at[0]], rows[0], gsem[0])
      for j in range(_G):
        p = j % 2
        if j + 1 < _G:
          if scat[1 - p] is not None:
            scat[1 - p].wait()
            scat[1 - p] = None
          gath[1 - p] = pltpu.async_copy(y.at[src_v.at[j + 1]], rows[1 - p],
                                         gsem[1 - p])
        gath[p].wait()
        scat[p] = pltpu.make_async_copy(rows[p], acc.at[dst_v.at[j]], ssem[p])
        scat[p].start(add=True)
        if with_counts:
          if cnt_cp is not None:
            cnt_cp.wait()
          cnt_cp = pltpu.make_async_copy(ones_v, cnt_sh.at[dst_v.at[j]], cs)
          cnt_cp.start(add=True)
      for p in range(2):
        if scat[p] is not None:
          scat[p].wait()
      if with_counts and cnt_cp is not None:
        cnt_cp.wait()
      return carry

    lax.fori_loop(0, n_groups, group, 0)
    plsc.subcore_barrier()

    wb = (n_nodes // _NS) // 8 * 8
    tail = n_nodes - _NS * wb
    pltpu.sync_copy(acc.at[pl.ds(sid * wb, wb)],
                    seg_out.at[cid, pl.ds(sid * wb, wb)])
    if with_counts:
      pltpu.sync_copy(cnt_sh.at[pl.ds(sid * wb, wb)],
                      cnt_out.at[cid, pl.ds(sid * wb, wb)])
    if tail:
      @pl.when(sid == 0)
      def _():
        pltpu.sync_copy(acc.at[pl.ds(_NS * wb, tail)],
                        seg_out.at[cid, pl.ds(_NS * wb, tail)])
        if with_counts:
          pltpu.sync_copy(cnt_sh.at[pl.ds(_NS * wb, tail)],
                          cnt_out.at[cid, pl.ds(_NS * wb, tail)])

  return pl.kernel(
      body, out_type=out_type, mesh=mesh, scratch_types=scratch,
      compiler_params=pltpu.CompilerParams(use_tc_tiling_on_sc=False))


def _proj2(x, Wa, Wb, b, block_n):
  n, d_in = x.shape
  d_out = Wa.shape[1]

  def body(x_ref, wa_ref, wb_ref, b_ref, ya_ref, yb_ref):
    xb = x_ref[...]
    ya_ref[...] = jnp.dot(xb, wa_ref[...], preferred_element_type=jnp.float32)
    yb_ref[...] = (jnp.dot(xb, wb_ref[...], preferred_element_type=jnp.float32)
                   + b_ref[...])

  return pl.pallas_call(
      body,
      grid=(n // block_n,),
      in_specs=[
          pl.BlockSpec((block_n, d_in), lambda i: (i, 0)),
          pl.BlockSpec((d_in, d_out), lambda i: (0, 0)),
          pl.BlockSpec((d_in, d_out), lambda i: (0, 0)),
          pl.BlockSpec((1, d_out), lambda i: (0, 0)),
      ],
      out_specs=[
          pl.BlockSpec((block_n, d_out), lambda i: (i, 0)),
          pl.BlockSpec((block_n, d_out), lambda i: (i, 0)),
      ],
      out_shape=[
          jax.ShapeDtypeStruct((n, d_out), jnp.float32),
          jax.ShapeDtypeStruct((n, d_out), jnp.float32),
      ],
  )(x, Wa, Wb, b.reshape(1, -1))


def _mid_layer(seg, cnt, z1, W2l, W2r, b2, block_n):
  _, n, d_h = seg.shape
  d_out = W2l.shape[1]

  def body(s_ref, c_ref, z1_ref, wl_ref, wr_ref, b_ref, y2_ref, z2_ref):
    c = jnp.maximum(c_ref[0, :, :1] + c_ref[1, :, :1], 1.0)
    mean = (s_ref[0] + s_ref[1]) / c
    h = jnp.maximum(mean + z1_ref[...], 0.0)
    y2_ref[...] = jnp.dot(h, wl_ref[...], preferred_element_type=jnp.float32)
    z2_ref[...] = (jnp.dot(h, wr_ref[...], preferred_element_type=jnp.float32)
                   + b_ref[...])

  return pl.pallas_call(
      body,
      grid=(n // block_n,),
      in_specs=[
          pl.BlockSpec((2, block_n, d_h), lambda i: (0, i, 0)),
          pl.BlockSpec((2, block_n, 16), lambda i: (0, i, 0)),
          pl.BlockSpec((block_n, d_h), lambda i: (i, 0)),
          pl.BlockSpec((d_h, d_out), lambda i: (0, 0)),
          pl.BlockSpec((d_h, d_out), lambda i: (0, 0)),
          pl.BlockSpec((1, d_out), lambda i: (0, 0)),
      ],
      out_specs=[
          pl.BlockSpec((block_n, d_out), lambda i: (i, 0)),
          pl.BlockSpec((block_n, d_out), lambda i: (i, 0)),
      ],
      out_shape=[
          jax.ShapeDtypeStruct((n, d_out), jnp.float32),
          jax.ShapeDtypeStruct((n, d_out), jnp.float32),
      ],
  )(seg, cnt, z1, W2l, W2r, b2.reshape(1, -1))


def _final_layer(seg, cnt, z2, block_n):
  _, n, d_out = seg.shape

  def body(s_ref, c_ref, z2_ref, out_ref):
    c = jnp.maximum(c_ref[0, :, :1] + c_ref[1, :, :1], 1.0)
    v = (s_ref[0] + s_ref[1]) / c + z2_ref[...]
    m = jnp.max(v, axis=1, keepdims=True)
    e = jnp.exp(v - m)
    s = jnp.sum(e, axis=1, keepdims=True)
    out_ref[...] = v - m - jnp.log(s)

  return pl.pallas_call(
      body,
      grid=(n // block_n,),
      in_specs=[
          pl.BlockSpec((2, block_n, d_out), lambda i: (0, i, 0)),
          pl.BlockSpec((2, block_n, 16), lambda i: (0, i, 0)),
          pl.BlockSpec((block_n, d_out), lambda i: (i, 0)),
      ],
      out_specs=pl.BlockSpec((block_n, d_out), lambda i: (i, 0)),
      out_shape=jax.ShapeDtypeStruct((n, d_out), jnp.float32),
  )(seg, cnt, z2)


def kernel(x, edge_index, W1l, W1r, b1, W2l, W2r, b2):
  n, d_in = x.shape
  e = edge_index.shape[1]
  d_h = W1l.shape[1]
  d_out = W2l.shape[1]

  epw = e // _NW
  n_chunks = epw // _C
  block_n = 1000

  src = edge_index[0].reshape(_NW, n_chunks, _C)
  dst = edge_index[1].reshape(_NW, n_chunks, _C)
  ones = jnp.ones((_C, 16), jnp.float32)
  zrow_h = jnp.zeros((n // _NS, d_h), jnp.float32)
  zrow_o = jnp.zeros((n // _NS, d_out), jnp.float32)
  zcnt = jnp.zeros((n // _NS, 16), jnp.float32)

  y1, z1 = _proj2(x, W1l, W1r, b1, block_n)
  seg1, cnt = _seg_sum_sc(d_h, n, n_chunks, True)(
      y1, src, dst, ones, zrow_h, zcnt)
  y2, z2 = _mid_layer(seg1, cnt, z1, W2l, W2r, b2, block_n)

  (seg2,) = _seg_sum_sc(d_out, n, n_chunks, False)(y2, src, dst, zrow_o)
  return _final_layer(seg2, cnt, z2, block_n)


# speedup vs baseline: 12.9518x; 1.0400x over previous
import functools

import jax
import jax.numpy as jnp
from jax import lax
from jax.experimental import pallas as pl
from jax.experimental.pallas import tpu as pltpu
from jax.experimental.pallas import tpu_sc as plsc

_NC = 2
_NS = 16
_NW = _NC * _NS

_C = 125
_G = 16


def _seg_sum_sc(d_model, n_nodes, n_chunks, with_counts):
  rpt = n_nodes // _NS
  n_groups = n_chunks // _G
  mesh = plsc.VectorSubcoreMesh(core_axis_name="c", subcore_axis_name="s")

  out_type = [jax.ShapeDtypeStruct((_NC, n_nodes, d_model), jnp.float32)]
  scratch = [
      pltpu.VMEM_SHARED((n_nodes, d_model), jnp.float32),
      pltpu.VMEM((_G, _C), jnp.int32),
      pltpu.VMEM((_G, _C), jnp.int32),
      pltpu.VMEM((_C, d_model), jnp.float32),
      pltpu.VMEM((_C, d_model), jnp.float32),
      pltpu.SemaphoreType.DMA,
      pltpu.SemaphoreType.DMA,
      pltpu.SemaphoreType.DMA,
      pltpu.SemaphoreType.DMA,
  ]
  if with_counts:
    out_type.append(jax.ShapeDtypeStruct((_NC, n_nodes, 16), jnp.float32))
    scratch += [
        pltpu.VMEM_SHARED((n_nodes, 16), jnp.float32),
        pltpu.VMEM((_C, 16), jnp.float32),
        pltpu.SemaphoreType.DMA,
    ]

  def body(*refs):
    if with_counts:
      (y, edges, ones, zrow, zcnt, seg_out, cnt_out,
       acc, src_v, dst_v, rows0, rows1, gs0, gs1, ss0, ss1,
       cnt_sh, ones_v, cs) = refs
    else:
      (y, edges, zrow, seg_out,
       acc, src_v, dst_v, rows0, rows1, gs0, gs1, ss0, ss1) = refs
    rows = (rows0, rows1)
    gsem = (gs0, gs1)
    ssem = (ss0, ss1)
    cid = lax.axis_index("c")
    sid = lax.axis_index("s")
    wid = sid * _NC + cid

    pltpu.sync_copy(zrow, acc.at[pl.ds(sid * rpt, rpt)])
    if with_counts:
      pltpu.sync_copy(ones, ones_v)
      pltpu.sync_copy(zcnt, cnt_sh.at[pl.ds(sid * rpt, rpt)])
    plsc.subcore_barrier()

    def group(g, carry):
      pltpu.sync_copy(edges.at[0, wid, pl.ds(g * _G, _G)], src_v)
      pltpu.sync_copy(edges.at[1, wid, pl.ds(g * _G, _G)], dst_v)

      gath = [None, None]
      scat = [None, None]
      cnt_cp = None
      gath[0] = pltpu.async_copy(y.at[src_v.at[0]], rows[0], gsem[0])
      for j in range(_G):
        p = j % 2
        if j + 1 < _G:
          if scat[1 - p] is not None:
            scat[1 - p].wait()
            scat[1 - p] = None
          gath[1 - p] = pltpu.async_copy(y.at[src_v.at[j + 1]], rows[1 - p],
                                         gsem[1 - p])
        gath[p].wait()
        scat[p] = pltpu.make_async_copy(rows[p], acc.at[dst_v.at[j]], ssem[p])
        scat[p].start(add=True)
        if with_counts:
          if cnt_cp is not None:
            cnt_cp.wait()
          cnt_cp = pltpu.make_async_copy(ones_v, cnt_sh.at[dst_v.at[j]], cs)
          cnt_cp.start(add=True)
      for p in range(2):
        if scat[p] is not None:
          scat[p].wait()
      if with_counts and cnt_cp is not None:
        cnt_cp.wait()
      return carry

    lax.fori_loop(0, n_groups, group, 0)
    plsc.subcore_barrier()

    wb = (n_nodes // _NS) // 8 * 8
    tail = n_nodes - _NS * wb
    pltpu.sync_copy(acc.at[pl.ds(sid * wb, wb)],
                    seg_out.at[cid, pl.ds(sid * wb, wb)])
    if with_counts:
      pltpu.sync_copy(cnt_sh.at[pl.ds(sid * wb, wb)],
                      cnt_out.at[cid, pl.ds(sid * wb, wb)])
    if tail:
      @pl.when(sid == 0)
      def _():
        pltpu.sync_copy(acc.at[pl.ds(_NS * wb, tail)],
                        seg_out.at[cid, pl.ds(_NS * wb, tail)])
        if with_counts:
          pltpu.sync_copy(cnt_sh.at[pl.ds(_NS * wb, tail)],
                          cnt_out.at[cid, pl.ds(_NS * wb, tail)])

  return pl.kernel(
      body, out_type=out_type, mesh=mesh, scratch_types=scratch,
      compiler_params=pltpu.CompilerParams(use_tc_tiling_on_sc=False))


def _proj(x, W, b, block_n):
  n, d_in = x.shape
  d_out = W.shape[1]

  def body(x_ref, w_ref, *rest):
    out_ref = rest[-1]
    r = jnp.dot(x_ref[...], w_ref[...], preferred_element_type=jnp.float32)
    if b is not None:
      r = r + rest[0][...]
    out_ref[...] = r

  in_specs = [
      pl.BlockSpec((block_n, d_in), lambda i: (i, 0)),
      pl.BlockSpec((d_in, d_out), lambda i: (0, 0)),
  ]
  args = [x, W]
  if b is not None:
    in_specs.append(pl.BlockSpec((1, d_out), lambda i: (0, 0)))
    args.append(b.reshape(1, -1))
  return pl.pallas_call(
      body,
      grid=(n // block_n,),
      in_specs=in_specs,
      out_specs=pl.BlockSpec((block_n, d_out), lambda i: (i, 0)),
      out_shape=jax.ShapeDtypeStruct((n, d_out), jnp.float32),
  )(*args)


def _mid_layer(seg, cnt, z1, W2l, W2r, b2, block_n):
  _, n, d_h = seg.shape
  d_out = W2l.shape[1]

  def body(s_ref, c_ref, z1_ref, wl_ref, wr_ref, b_ref, y2_ref, z2_ref):
    c = jnp.maximum(c_ref[0, :, :1] + c_ref[1, :, :1], 1.0)
    mean = (s_ref[0] + s_ref[1]) / c
    h = jnp.maximum(mean + z1_ref[...], 0.0)
    y2_ref[...] = jnp.dot(h, wl_ref[...], preferred_element_type=jnp.float32)
    z2_ref[...] = (jnp.dot(h, wr_ref[...], preferred_element_type=jnp.float32)
                   + b_ref[...])

  return pl.pallas_call(
      body,
      grid=(n // block_n,),
      in_specs=[
          pl.BlockSpec((2, block_n, d_h), lambda i: (0, i, 0)),
          pl.BlockSpec((2, block_n, 16), lambda i: (0, i, 0)),
          pl.BlockSpec((block_n, d_h), lambda i: (i, 0)),
          pl.BlockSpec((d_h, d_out), lambda i: (0, 0)),
          pl.BlockSpec((d_h, d_out), lambda i: (0, 0)),
          pl.BlockSpec((1, d_out), lambda i: (0, 0)),
      ],
      out_specs=[
          pl.BlockSpec((block_n, d_out), lambda i: (i, 0)),
          pl.BlockSpec((block_n, d_out), lambda i: (i, 0)),
      ],
      out_shape=[
          jax.ShapeDtypeStruct((n, d_out), jnp.float32),
          jax.ShapeDtypeStruct((n, d_out), jnp.float32),
      ],
  )(seg, cnt, z1, W2l, W2r, b2.reshape(1, -1))


def _final_layer(seg, cnt, z2, block_n):
  _, n, d_out = seg.shape

  def body(s_ref, c_ref, z2_ref, out_ref):
    c = jnp.maximum(c_ref[0, :, :1] + c_ref[1, :, :1], 1.0)
    v = (s_ref[0] + s_ref[1]) / c + z2_ref[...]
    m = jnp.max(v, axis=1, keepdims=True)
    e = jnp.exp(v - m)
    s = jnp.sum(e, axis=1, keepdims=True)
    out_ref[...] = v - m - jnp.log(s)

  return pl.pallas_call(
      body,
      grid=(n // block_n,),
      in_specs=[
          pl.BlockSpec((2, block_n, d_out), lambda i: (0, i, 0)),
          pl.BlockSpec((2, block_n, 16), lambda i: (0, i, 0)),
          pl.BlockSpec((block_n, d_out), lambda i: (i, 0)),
      ],
      out_specs=pl.BlockSpec((block_n, d_out), lambda i: (i, 0)),
      out_shape=jax.ShapeDtypeStruct((n, d_out), jnp.float32),
  )(seg, cnt, z2)


def kernel(x, edge_index, W1l, W1r, b1, W2l, W2r, b2):
  n, d_in = x.shape
  e = edge_index.shape[1]
  d_h = W1l.shape[1]
  d_out = W2l.shape[1]

  epw = e // _NW
  n_chunks = epw // _C
  block_n = 1000

  e4 = edge_index.reshape(2, _NW, n_chunks, _C)
  ones = jnp.ones((_C, 16), jnp.float32)
  zrow_h = jnp.zeros((n // _NS, d_h), jnp.float32)
  zrow_o = jnp.zeros((n // _NS, d_out), jnp.float32)
  zcnt = jnp.zeros((n // _NS, 16), jnp.float32)

  y1 = _proj(x, W1l, None, block_n)
  seg1, cnt = _seg_sum_sc(d_h, n, n_chunks, True)(
      y1, e4, ones, zrow_h, zcnt)
  z1 = _proj(x, W1r, b1, block_n)
  y2, z2 = _mid_layer(seg1, cnt, z1, W2l, W2r, b2, block_n)

  (seg2,) = _seg_sum_sc(d_out, n, n_chunks, False)(y2, e4, zrow_o)
  return _final_layer(seg2, cnt, z2, block_n)
